# baseline (speedup 1.0000x reference)
import functools

import jax
import jax.numpy as jnp
from jax import lax
from jax.experimental import pallas as pl
from jax.experimental.pallas import tpu as pltpu
from jax.experimental.pallas import tpu_sc as plsc


def _sc_gaines_div(n_total, per_w, chunk, num_cores):
    nch = per_w // chunk
    nv = chunk // 16
    unroll = 8

    mesh = plsc.VectorSubcoreMesh(core_axis_name="c", subcore_axis_name="s")

    @functools.partial(
        pl.kernel,
        mesh=mesh,
        out_type=jax.ShapeDtypeStruct((n_total,), jnp.float32),
        scratch_types=[
            pltpu.VMEM((chunk,), jnp.float32),
            pltpu.VMEM((chunk,), jnp.float32),
            pltpu.VMEM((chunk,), jnp.float32),
        ],
    )
    def sc_k(d_hbm, out_hbm, a, b, o):
        wid = lax.axis_index("s") * num_cores + lax.axis_index("c")
        base = wid * per_w

        def chunk_body(j, carry):
            off = base + j * chunk
            pltpu.sync_copy(d_hbm.at[pl.ds(off, chunk)], a)
            pltpu.sync_copy(d_hbm.at[pl.ds(n_total + off, chunk)], b)

            def vec_body(g, c2):
                for u in range(unroll):
                    k16 = (g * unroll + u) * 16
                    s = a[pl.ds(k16, 16)] + b[pl.ds(k16, 16)]
                    o[pl.ds(k16, 16)] = jnp.where(s > 0.0, 1.0, 0.0)
                return c2

            lax.fori_loop(0, nv // unroll, vec_body, 0)
            pltpu.sync_copy(o, out_hbm.at[pl.ds(off, chunk)])
            return carry

        lax.fori_loop(0, nch, chunk_body, 0)

    return sc_k


def kernel(dividend, divisor):
    del divisor
    _, rows, cols = dividend.shape
    n = rows * cols
    info = plsc.get_sparse_core_info()
    nw = info.num_cores * info.num_subcores
    per_w = n // nw
    chunk = 16384
    flat = dividend.reshape(2 * n)
    out = _sc_gaines_div(n, per_w, chunk, info.num_cores)(flat)
    return out.reshape(rows, cols)


# speedup vs baseline: 1.1460x; 1.1460x over previous
import functools

import jax
import jax.numpy as jnp
from jax import lax
from jax.experimental import pallas as pl
from jax.experimental.pallas import tpu as pltpu
from jax.experimental.pallas import tpu_sc as plsc


def _sc_gaines_div(n_total, per_w, chunk, num_cores):
    nch = per_w // chunk
    assert nch % 2 == 0

    mesh = plsc.VectorSubcoreMesh(core_axis_name="c", subcore_axis_name="s")

    @functools.partial(
        pl.kernel,
        mesh=mesh,
        out_type=jax.ShapeDtypeStruct((n_total,), jnp.float32),
        scratch_types=[
            pltpu.VMEM((2, chunk), jnp.float32),
            pltpu.VMEM((2, chunk), jnp.float32),
            pltpu.VMEM((2, chunk), jnp.float32),
            pltpu.SemaphoreType.DMA((2,)),
            pltpu.SemaphoreType.DMA((2,)),
        ],
    )
    def sc_k(d_hbm, out_hbm, va, vb, vo, sem_in, sem_out):
        wid = lax.axis_index("s") * num_cores + lax.axis_index("c")
        base = wid * per_w

        def start_in(j, slot):
            off = base + j * chunk
            pltpu.async_copy(d_hbm.at[pl.ds(off, chunk)], va.at[slot],
                             sem_in.at[slot])
            pltpu.async_copy(d_hbm.at[pl.ds(n_total + off, chunk)],
                             vb.at[slot], sem_in.at[slot])

        def wait_in(slot):
            pltpu.make_async_copy(d_hbm.at[pl.ds(base, chunk)], va.at[slot],
                                  sem_in.at[slot]).wait()
            pltpu.make_async_copy(d_hbm.at[pl.ds(base, chunk)], vb.at[slot],
                                  sem_in.at[slot]).wait()

        def wait_out(slot):
            pltpu.make_async_copy(vo.at[slot], out_hbm.at[pl.ds(base, chunk)],
                                  sem_out.at[slot]).wait()

        start_in(0, 0)
        start_in(1, 1)

        def step(g, carry):
            for slot in range(2):
                j = g * 2 + slot
                wait_in(slot)

                @pl.when(g > 0)
                def _():
                    wait_out(slot)

                aref, bref, oref = va.at[slot], vb.at[slot], vo.at[slot]

                @plsc.parallel_loop(0, chunk, 16, unroll=8)
                def _(k):
                    s = aref[pl.ds(k, 16)] + bref[pl.ds(k, 16)]
                    oref[pl.ds(k, 16)] = jnp.where(s > 0.0, 1.0, 0.0)

                pltpu.async_copy(vo.at[slot],
                                 out_hbm.at[pl.ds(base + j * chunk, chunk)],
                                 sem_out.at[slot])

                @pl.when(j + 2 < nch)
                def _():
                    start_in(j + 2, slot)
            return carry

        lax.fori_loop(0, nch // 2, step, 0)
        wait_out(0)
        wait_out(1)

    return sc_k


def kernel(dividend, divisor):
    del divisor
    _, rows, cols = dividend.shape
    n = rows * cols
    info = plsc.get_sparse_core_info()
    nw = info.num_cores * info.num_subcores
    per_w = n // nw
    chunk = 16384
    flat = dividend.reshape(2 * n)
    out = _sc_gaines_div(n, per_w, chunk, info.num_cores)(flat)
    return out.reshape(rows, cols)


# speedup vs baseline: 3.0765x; 2.6844x over previous
import functools

import jax
import jax.numpy as jnp
from jax import lax
from jax.experimental import pallas as pl
from jax.experimental.pallas import tpu as pltpu
from jax.experimental.pallas import tpu_sc as plsc


def _sc_gaines_div(rows, cols, rows_per_w, stripe, num_cores):
    nch = rows_per_w // stripe
    assert nch % 2 == 0

    mesh = plsc.VectorSubcoreMesh(core_axis_name="c", subcore_axis_name="s")

    @functools.partial(
        pl.kernel,
        mesh=mesh,
        out_type=jax.ShapeDtypeStruct((rows, cols), jnp.float32),
        scratch_types=[
            pltpu.VMEM((2, stripe, cols), jnp.float32),
            pltpu.VMEM((2, stripe, cols), jnp.float32),
            pltpu.VMEM((2, stripe, cols), jnp.float32),
            pltpu.SemaphoreType.DMA((2,)),
            pltpu.SemaphoreType.DMA((2,)),
        ],
    )
    def sc_k(d_hbm, out_hbm, va, vb, vo, sem_in, sem_out):
        wid = lax.axis_index("s") * num_cores + lax.axis_index("c")
        base = wid * rows_per_w

        def start_in(j, slot):
            row = base + j * stripe
            pltpu.async_copy(d_hbm.at[0, pl.ds(row, stripe), :], va.at[slot],
                             sem_in.at[slot])
            pltpu.async_copy(d_hbm.at[1, pl.ds(row, stripe), :], vb.at[slot],
                             sem_in.at[slot])

        def wait_in(slot):
            pltpu.make_async_copy(d_hbm.at[0, pl.ds(base, stripe), :],
                                  va.at[slot], sem_in.at[slot]).wait()
            pltpu.make_async_copy(d_hbm.at[0, pl.ds(base, stripe), :],
                                  vb.at[slot], sem_in.at[slot]).wait()

        def wait_out(slot):
            pltpu.make_async_copy(vo.at[slot],
                                  out_hbm.at[pl.ds(base, stripe), :],
                                  sem_out.at[slot]).wait()

        start_in(0, 0)
        start_in(1, 1)

        def step(g, carry):
            for slot in range(2):
                j = g * 2 + slot
                wait_in(slot)

                @pl.when(g > 0)
                def _():
                    wait_out(slot)

                for r in range(stripe):

                    @plsc.parallel_loop(0, cols, 16, unroll=8)
                    def _(k):
                        s = va[slot, r, pl.ds(k, 16)] + vb[slot, r, pl.ds(k, 16)]
                        vo[slot, r, pl.ds(k, 16)] = jnp.where(s > 0.0, 1.0, 0.0)

                pltpu.async_copy(
                    vo.at[slot],
                    out_hbm.at[pl.ds(base + j * stripe, stripe), :],
                    sem_out.at[slot])

                @pl.when(j + 2 < nch)
                def _():
                    start_in(j + 2, slot)
            return carry

        lax.fori_loop(0, nch // 2, step, 0)
        wait_out(0)
        wait_out(1)

    return sc_k


def kernel(dividend, divisor):
    del divisor
    _, rows, cols = dividend.shape
    info = plsc.get_sparse_core_info()
    nw = info.num_cores * info.num_subcores
    rows_per_w = rows // nw
    stripe = 8
    out = _sc_gaines_div(rows, cols, rows_per_w, stripe, info.num_cores)(dividend)
    return out
